# Initial kernel scaffold; baseline (speedup 1.0000x reference)
import jax
import jax.numpy as jnp
from jax.experimental import pallas as pl


def kernel(x, edge_index, W1, W2):
    raise NotImplementedError("write your pallas kernel here")


# speedup vs baseline: 5.6518x; 5.6518x over previous
import functools

import jax
import jax.numpy as jnp
from jax import lax
from jax.experimental import pallas as pl
from jax.experimental.pallas import tpu as pltpu
from jax.experimental.pallas import tpu_sc as plsc

N_NODES = 10000
N_EDGES = 160000
IN_CH = 256
HIDDEN = 512
OUT_CH = 128

NC = 2
NS = 16
NW = NC * NS
CHUNK = 128
N_CHUNKS = N_EDGES // CHUNK
CHUNKS_PER_TILE = -(-N_CHUNKS // NW)
ROWS_A = 624
ROWS_LAST = N_NODES - (NS - 1) * ROWS_A


def _mlp_body(x_ref, w1_ref, w2_ref, h_ref):
    t = jnp.dot(x_ref[...], w1_ref[...], preferred_element_type=jnp.float32)
    t = jnp.maximum(t, 0.0)
    h_ref[...] = jnp.dot(t, w2_ref[...], preferred_element_type=jnp.float32)


def _mlp(x, W1, W2):
    R = 1000
    return pl.pallas_call(
        _mlp_body,
        grid=(N_NODES // R,),
        in_specs=[
            pl.BlockSpec((R, IN_CH), lambda i: (i, 0)),
            pl.BlockSpec((IN_CH, HIDDEN), lambda i: (0, 0)),
            pl.BlockSpec((HIDDEN, OUT_CH), lambda i: (0, 0)),
        ],
        out_specs=pl.BlockSpec((R, OUT_CH), lambda i: (i, 0)),
        out_shape=jax.ShapeDtypeStruct((N_NODES, OUT_CH), jnp.float32),
    )(x, W1, W2)


def _sc_body(h_hbm, src_hbm, dst_hbm, out_hbm,
             acc, rows, src_v, dst_v, sem):
    cid = lax.axis_index("c")
    sid = lax.axis_index("s")
    wid = sid * NC + cid
    rbase = pl.multiple_of(sid * ROWS_A, 8)

    def zrow(i, carry):
        def zcol(j, c2):
            rows[i, pl.ds(j * 16, 16)] = jnp.zeros((16,), jnp.float32)
            return c2
        return lax.fori_loop(0, OUT_CH // 16, zcol, carry)
    lax.fori_loop(0, CHUNK, zrow, 0)

    @pl.when(sid < NS - 1)
    def _():
        for k in range(ROWS_A // CHUNK):
            pltpu.sync_copy(rows, acc.at[pl.ds(rbase + k * CHUNK, CHUNK)])
        tail = ROWS_A % CHUNK
        pltpu.sync_copy(rows.at[pl.ds(0, tail)],
                        acc.at[pl.ds(rbase + ROWS_A - tail, tail)])

    @pl.when(sid == NS - 1)
    def _():
        for k in range(ROWS_LAST // CHUNK):
            pltpu.sync_copy(rows, acc.at[pl.ds(rbase + k * CHUNK, CHUNK)])
    plsc.subcore_barrier()

    def step(i, carry):
        chunk_id = i * NW + wid

        @pl.when(chunk_id < N_CHUNKS)
        def _():
            base = chunk_id * CHUNK
            pltpu.sync_copy(src_hbm.at[pl.ds(base, CHUNK)], src_v)
            pltpu.sync_copy(dst_hbm.at[pl.ds(base, CHUNK)], dst_v)
            pltpu.async_copy(h_hbm.at[src_v], rows, sem).wait()
            pltpu.sync_copy(rows, acc.at[dst_v], add=True)
        return carry
    lax.fori_loop(0, CHUNKS_PER_TILE, step, 0)

    plsc.subcore_barrier()

    @pl.when(sid < NS - 1)
    def _():
        for k in range(ROWS_A // CHUNK):
            sl = pl.ds(rbase + k * CHUNK, CHUNK)
            pltpu.sync_copy(acc.at[sl], rows)
            pltpu.sync_copy(rows, out_hbm.at[cid, sl])
        tail = ROWS_A % CHUNK
        sl = pl.ds(rbase + ROWS_A - tail, tail)
        pltpu.sync_copy(acc.at[sl], rows.at[pl.ds(0, tail)])
        pltpu.sync_copy(rows.at[pl.ds(0, tail)], out_hbm.at[cid, sl])

    @pl.when(sid == NS - 1)
    def _():
        for k in range(ROWS_LAST // CHUNK):
            sl = pl.ds(rbase + k * CHUNK, CHUNK)
            pltpu.sync_copy(acc.at[sl], rows)
            pltpu.sync_copy(rows, out_hbm.at[cid, sl])


def _sc_scatter(h, src, dst):
    mesh = plsc.VectorSubcoreMesh(core_axis_name="c", subcore_axis_name="s")
    fn = pl.kernel(
        _sc_body,
        out_type=jax.ShapeDtypeStruct((NC, N_NODES, OUT_CH), jnp.float32),
        mesh=mesh,
        scratch_types=[
            pltpu.VMEM_SHARED((N_NODES, OUT_CH), jnp.float32),
            pltpu.VMEM((CHUNK, OUT_CH), jnp.float32),
            pltpu.VMEM((CHUNK,), jnp.int32),
            pltpu.VMEM((CHUNK,), jnp.int32),
            pltpu.SemaphoreType.DMA,
        ],
    )
    return fn(h, src, dst)


def _add_body(p_ref, o_ref):
    o_ref[...] = p_ref[0] + p_ref[1]


def _combine(partials):
    R = 2000
    return pl.pallas_call(
        _add_body,
        grid=(N_NODES // R,),
        in_specs=[pl.BlockSpec((NC, R, OUT_CH), lambda i: (0, i, 0))],
        out_specs=pl.BlockSpec((R, OUT_CH), lambda i: (i, 0)),
        out_shape=jax.ShapeDtypeStruct((N_NODES, OUT_CH), jnp.float32),
    )(partials)


def kernel(x, edge_index, W1, W2):
    h = _mlp(x, W1, W2)
    src = edge_index[0].astype(jnp.int32)
    dst = edge_index[1].astype(jnp.int32)
    partials = _sc_scatter(h, src, dst)
    return _combine(partials)
